# Initial kernel scaffold; baseline (speedup 1.0000x reference)
import jax
import jax.numpy as jnp
from jax.experimental import pallas as pl


def kernel(w):
    raise NotImplementedError("write your pallas kernel here")


# speedup vs baseline: 5.7915x; 5.7915x over previous
import jax
import jax.numpy as jnp
from jax.experimental import pallas as pl

_SELECTED = (0, 2, 4, 6, 8, 10, 12, 14)
_ROWS = 512


def _copy_body(in_ref, out_ref):
    out_ref[...] = in_ref[...]


def kernel(w):
    n_out = len(_SELECTED)
    _, H, W = w.shape
    return pl.pallas_call(
        _copy_body,
        grid=(n_out, H // _ROWS),
        in_specs=[pl.BlockSpec((1, _ROWS, W), lambda i, j: (2 * i, j, 0))],
        out_specs=pl.BlockSpec((1, _ROWS, W), lambda i, j: (i, j, 0)),
        out_shape=jax.ShapeDtypeStruct((n_out, H, W), w.dtype),
    )(w)


# speedup vs baseline: 5.9250x; 1.0231x over previous
import jax
import jax.numpy as jnp
from jax.experimental import pallas as pl

_SELECTED = (0, 2, 4, 6, 8, 10, 12, 14)
_ROWS = 1024


def _copy_body(in_ref, out_ref):
    out_ref[...] = in_ref[...]


def kernel(w):
    n_out = len(_SELECTED)
    _, H, W = w.shape
    return pl.pallas_call(
        _copy_body,
        grid=(n_out, H // _ROWS),
        in_specs=[pl.BlockSpec((1, _ROWS, W), lambda i, j: (2 * i, j, 0))],
        out_specs=pl.BlockSpec((1, _ROWS, W), lambda i, j: (i, j, 0)),
        out_shape=jax.ShapeDtypeStruct((n_out, H, W), w.dtype),
    )(w)
